# Initial kernel scaffold; baseline (speedup 1.0000x reference)
import jax
import jax.numpy as jnp
from jax.experimental import pallas as pl


def kernel(token_ids, W):
    raise NotImplementedError("write your pallas kernel here")


# speedup vs baseline: 4.5084x; 4.5084x over previous
import functools

import jax
import jax.numpy as jnp
from jax import lax
from jax.experimental import pallas as pl
from jax.experimental.pallas import tpu as pltpu
from jax.experimental.pallas import tpu_sc as plsc

_NC = 2
_NS = 16
_NW = _NC * _NS

_CHUNK = 128


@functools.partial(jax.jit, static_argnames=("b", "d"))
def _sc_gather(table, idx, *, b, d):
    per_w = b // _NW
    n_chunks = per_w // _CHUNK
    mesh = plsc.VectorSubcoreMesh(core_axis_name="c", subcore_axis_name="s")

    @functools.partial(
        pl.kernel,
        out_type=jax.ShapeDtypeStruct((b, d), jnp.float32),
        mesh=mesh,
        scratch_types=[
            pltpu.VMEM((_CHUNK,), jnp.int32),
            pltpu.VMEM((_CHUNK, d), jnp.float32),
            pltpu.SemaphoreType.DMA,
        ],
        compiler_params=pltpu.CompilerParams(use_tc_tiling_on_sc=False),
    )
    def k(table_hbm, idx_hbm, out_hbm, idx_v, rows_v, sem):
        wid = lax.axis_index("s") * _NC + lax.axis_index("c")
        base = wid * per_w

        def chunk_body(g, carry):
            off = base + g * _CHUNK
            pltpu.sync_copy(idx_hbm.at[pl.ds(off, _CHUNK)], idx_v)
            pltpu.async_copy(table_hbm.at[idx_v], rows_v, sem).wait()
            pltpu.sync_copy(rows_v, out_hbm.at[pl.ds(off, _CHUNK)])
            return carry

        lax.fori_loop(0, n_chunks, chunk_body, 0)

    return k(table, idx)


def kernel(token_ids, W):
    shape = token_ids.shape
    d = W.shape[1]
    idx = token_ids.reshape(-1).astype(jnp.int32)
    out = _sc_gather(W, idx, b=idx.shape[0], d=d)
    return out.reshape(*shape, d)


# speedup vs baseline: 5.6501x; 1.2533x over previous
import functools

import jax
import jax.numpy as jnp
from jax import lax
from jax.experimental import pallas as pl
from jax.experimental.pallas import tpu as pltpu
from jax.experimental.pallas import tpu_sc as plsc

_NC = 2
_NS = 16
_NW = _NC * _NS

_CHUNK = 128
_NBUF = 8


@functools.partial(jax.jit, static_argnames=("b", "d"))
def _sc_gather(table, idx2d, *, b, d):
    per_w = b // _NW
    n_chunks = per_w // _CHUNK
    n_blocks = n_chunks // _NBUF
    mesh = plsc.VectorSubcoreMesh(core_axis_name="c", subcore_axis_name="s")

    @functools.partial(
        pl.kernel,
        out_type=jax.ShapeDtypeStruct((b, d), jnp.float32),
        mesh=mesh,
        scratch_types=[
            pltpu.VMEM((n_chunks, _CHUNK), jnp.int32),
            pltpu.VMEM((_NBUF, _CHUNK, d), jnp.float32),
            pltpu.SemaphoreType.DMA((_NBUF,)),
            pltpu.SemaphoreType.DMA((_NBUF,)),
        ],
        compiler_params=pltpu.CompilerParams(use_tc_tiling_on_sc=False),
    )
    def k(table_hbm, idx_hbm, out_hbm, idx_v, rows_v, sem_g, sem_w):
        wid = lax.axis_index("s") * _NC + lax.axis_index("c")
        base = wid * per_w

        pltpu.sync_copy(idx_hbm.at[pl.ds(wid * n_chunks, n_chunks)], idx_v)

        def start_gather(g, slot):
            pltpu.async_copy(table_hbm.at[idx_v.at[g]], rows_v.at[slot],
                             sem_g.at[slot])

        def start_write(g, slot):
            off = base + g * _CHUNK
            pltpu.async_copy(rows_v.at[slot], out_hbm.at[pl.ds(off, _CHUNK)],
                             sem_w.at[slot])

        def wait_gather(g, slot):
            pltpu.make_async_copy(table_hbm.at[idx_v.at[g]], rows_v.at[slot],
                                  sem_g.at[slot]).wait()

        def wait_write(g, slot):
            off = base + g * _CHUNK
            pltpu.make_async_copy(rows_v.at[slot],
                                  out_hbm.at[pl.ds(off, _CHUNK)],
                                  sem_w.at[slot]).wait()

        for s in range(_NBUF):
            start_gather(s, s)
        for s in range(_NBUF):
            wait_gather(s, s)
            start_write(s, s)

        def block_body(j, carry):
            g0 = j * _NBUF
            for s in range(_NBUF):
                wait_write(g0 - _NBUF + s, s)
                start_gather(g0 + s, s)
            for s in range(_NBUF):
                wait_gather(g0 + s, s)
                start_write(g0 + s, s)
            return carry

        lax.fori_loop(1, n_blocks, block_body, 0)

        g0 = (n_blocks - 1) * _NBUF
        for s in range(_NBUF):
            wait_write(g0 + s, s)

    return k(table, idx2d)


def kernel(token_ids, W):
    shape = token_ids.shape
    d = W.shape[1]
    idx = token_ids.reshape(-1, _CHUNK).astype(jnp.int32)
    out = _sc_gather(W, idx, b=token_ids.size, d=d)
    return out.reshape(*shape, d)
